# baseline (speedup 1.0000x reference)
import functools

import jax
import jax.numpy as jnp
from jax import lax
from jax.experimental import pallas as pl
from jax.experimental.pallas import tpu as pltpu
from jax.experimental.pallas import tpu_sc as plsc

N = 10000
E = 160000
DIN = 256
DH = 256
DOUT = 128
NB = 64
HD = 128

E_TOT = E + N
NBLK = 96
CE = 128
T_E = NBLK * CE
E_PAD = 16 * T_E
EROWS = E_PAD // CE
SROWS = 80
NP = 10240
PASS_R = 5120
NPASS = 2
ACC_R = PASS_R + 128
NEG = -1e30

_mesh = plsc.VectorSubcoreMesh(core_axis_name="c", subcore_axis_name="s")
_scp = pltpu.CompilerParams(use_tc_tiling_on_sc=False,
                            needs_layout_passes=False)


def _tc_in_body(x_ref, w_ref, as_ref, ad_ref, hh_ref, sc_ref):
    h = lax.dot_general(x_ref[...], w_ref[...], (((1,), (1,)), ((), ())),
                        preferred_element_type=jnp.float32)
    hh_ref[0] = h[:, :HD]
    hh_ref[1] = h[:, HD:]
    s0 = jnp.sum(h * as_ref[...], axis=1, keepdims=True)
    s1 = jnp.sum(h * ad_ref[...], axis=1, keepdims=True)
    sc_ref[...] = jnp.concatenate([s0, s1], axis=1)


_BM = 1000


def _tc_in(x, W, a_s, a_d):
    return pl.pallas_call(
        _tc_in_body,
        grid=(N // _BM,),
        in_specs=[
            pl.BlockSpec((_BM, DIN), lambda i: (i, 0)),
            pl.BlockSpec((DH, DIN), lambda i: (0, 0)),
            pl.BlockSpec((1, DH), lambda i: (0, 0)),
            pl.BlockSpec((1, DH), lambda i: (0, 0)),
        ],
        out_specs=[
            pl.BlockSpec((2, _BM, HD), lambda i: (0, i, 0)),
            pl.BlockSpec((_BM, 2), lambda i: (i, 0)),
        ],
        out_shape=[
            jax.ShapeDtypeStruct((2, NP, HD), jnp.float32),
            jax.ShapeDtypeStruct((NP, 2), jnp.float32),
        ],
    )(x, W, a_s, a_d)


def _tc_mid_body(a0_ref, a1_ref, b_ref, w_ref, as_ref, ad_ref,
                 hh_ref, sc_ref):
    x0 = jnp.maximum(a0_ref[...] + b_ref[:, :HD], 0.0)
    x1 = jnp.maximum(a1_ref[...] + b_ref[:, HD:], 0.0)
    h = (lax.dot_general(x0, w_ref[:, :HD], (((1,), (1,)), ((), ())),
                         preferred_element_type=jnp.float32)
         + lax.dot_general(x1, w_ref[:, HD:], (((1,), (1,)), ((), ())),
                           preferred_element_type=jnp.float32))
    hh_ref[0] = h[:, :HD]
    hh_ref[1] = h[:, HD:]
    s0 = jnp.sum(h * as_ref[...], axis=1, keepdims=True)
    s1 = jnp.sum(h * ad_ref[...], axis=1, keepdims=True)
    sc_ref[...] = jnp.concatenate([s0, s1], axis=1)


_BM2 = 1024


def _tc_mid(agg, b, W, a_s, a_d):
    nb2 = NP // _BM2
    return pl.pallas_call(
        _tc_mid_body,
        grid=(nb2,),
        in_specs=[
            pl.BlockSpec((_BM2, HD), lambda i: (i, 0)),
            pl.BlockSpec((_BM2, HD), lambda i, nb2=nb2: (i + nb2, 0)),
            pl.BlockSpec((1, DH), lambda i: (0, 0)),
            pl.BlockSpec((DH, DH), lambda i: (0, 0)),
            pl.BlockSpec((1, DH), lambda i: (0, 0)),
            pl.BlockSpec((1, DH), lambda i: (0, 0)),
        ],
        out_specs=[
            pl.BlockSpec((2, _BM2, HD), lambda i: (0, i, 0)),
            pl.BlockSpec((_BM2, 2), lambda i: (i, 0)),
        ],
        out_shape=[
            jax.ShapeDtypeStruct((2, NP, HD), jnp.float32),
            jax.ShapeDtypeStruct((NP, 2), jnp.float32),
        ],
    )(agg, agg, b, W, a_s, a_d)


_BMF = 512


def _tc_fin_body(a0_ref, a1_ref, b_ref, bt_ref, wf_ref, bf_ref, o_ref,
                 acc_ref, cnt_ref):
    i = pl.program_id(0)

    @pl.when(i == 0)
    def _():
        acc_ref[...] = jnp.zeros_like(acc_ref)
        cnt_ref[...] = jnp.zeros_like(cnt_ref)

    x0 = jnp.maximum(a0_ref[...] + b_ref[:, :HD], 0.0)
    x1 = jnp.maximum(a1_ref[...] + b_ref[:, HD:], 0.0)
    bt = bt_ref[0]
    oh = (lax.broadcasted_iota(jnp.int32, (NB, _BMF), 0) == bt
          ).astype(jnp.float32)
    acc_ref[...] += jnp.concatenate(
        [lax.dot_general(oh, x0, (((1,), (0,)), ((), ())),
                         preferred_element_type=jnp.float32),
         lax.dot_general(oh, x1, (((1,), (0,)), ((), ())),
                         preferred_element_type=jnp.float32)], axis=1)
    cnt_ref[...] += lax.dot_general(oh, jnp.ones((_BMF, HD), jnp.float32),
                                    (((1,), (0,)), ((), ())),
                                    preferred_element_type=jnp.float32)

    @pl.when(i == pl.num_programs(0) - 1)
    def _():
        pooled = acc_ref[...] / jnp.maximum(cnt_ref[:, :1], 1.0)
        o_ref[...] = lax.dot_general(pooled, wf_ref[...],
                                     (((1,), (1,)), ((), ())),
                                     preferred_element_type=jnp.float32
                                     ) + bf_ref[...]


def _tc_fin(agg, b, batch3, Wf, bf):
    nbf = NP // _BMF
    return pl.pallas_call(
        _tc_fin_body,
        grid=(nbf,),
        in_specs=[
            pl.BlockSpec((_BMF, HD), lambda i: (i, 0)),
            pl.BlockSpec((_BMF, HD), lambda i, nbf=nbf: (i + nbf, 0)),
            pl.BlockSpec((1, DH), lambda i: (0, 0)),
            pl.BlockSpec((1, 1, _BMF), lambda i: (i, 0, 0)),
            pl.BlockSpec((DOUT, DH), lambda i: (0, 0)),
            pl.BlockSpec((1, DOUT), lambda i: (0, 0)),
        ],
        out_specs=pl.BlockSpec((NB, DOUT), lambda i: (0, 0)),
        out_shape=jax.ShapeDtypeStruct((NB, DOUT), jnp.float32),
        scratch_shapes=[
            pltpu.VMEM((NB, DH), jnp.float32),
            pltpu.VMEM((NB, HD), jnp.float32),
        ],
    )(agg, agg, b, batch3, Wf, bf)


@functools.partial(
    pl.kernel,
    out_type=jax.ShapeDtypeStruct((EROWS, CE), jnp.float32),
    mesh=_mesh,
    compiler_params=_scp,
    scratch_types=[
        pltpu.VMEM((NP, 2), jnp.float32),
        pltpu.VMEM((NBLK, CE), jnp.int32),
        pltpu.VMEM((NBLK, CE), jnp.int32),
        pltpu.VMEM((NBLK, CE), jnp.float32),
        pltpu.VMEM((SROWS, CE), jnp.float32),
        pltpu.VMEM((16,), jnp.float32),
        pltpu.VMEM((16, 16), jnp.float32),
        pltpu.VMEM((SROWS,), jnp.int32),
        pltpu.VMEM((NBLK,), jnp.int32),
        pltpu.SemaphoreType.DMA,
        pltpu.VMEM_SHARED((16, 16), jnp.float32),
        pltpu.VMEM_SHARED((SROWS, CE), jnp.float32),
    ],
)
def _sc_softmax(sc_hbm, src_hbm, dst_hbm, alpha_hbm,
                sc_v, src_v, dst_v, e_v, s_v, mx_v, mx16_v, rid_v, crid_v,
                sem, mx_sh, s_sh):
    sid = lax.axis_index("s")
    cid = lax.axis_index("c")
    row0 = sid * NBLK

    pltpu.sync_copy(sc_hbm, sc_v)

    def _cr(i, _):
        crid_v[pl.ds(i * 16, 16)] = row0 + i * 16 + lax.iota(jnp.int32, 16)
        return 0
    lax.fori_loop(0, NBLK // 16, _cr, 0)
    pltpu.async_copy(src_hbm.at[crid_v], src_v, sem).wait()
    pltpu.async_copy(dst_hbm.at[crid_v], dst_v, sem).wait()

    def _z(i, _):
        s_v[i // 8, pl.ds((i % 8) * 16, 16)] = jnp.zeros((16,), jnp.float32)
        return 0
    lax.fori_loop(0, SROWS * 8, _z, 0)

    def _r(i, _):
        rid_v[pl.ds(i * 16, 16)] = i * 16 + lax.iota(jnp.int32, 16)
        return 0
    lax.fori_loop(0, SROWS // 16, _r, 0)

    @pl.when(sid == 0)
    def _():
        pltpu.sync_copy(s_v, s_sh)

    ebase = sid * T_E

    def _e(i, vmax):
        j = i // 8
        k = i % 8
        sl = pl.ds(k * 16, 16)
        isrc = src_v[j, sl]
        idst = dst_v[j, sl]
        zi = jnp.zeros((16,), jnp.int32)
        e = (plsc.load_gather(sc_v, [isrc, zi])
             + plsc.load_gather(sc_v, [idst, zi + 1]))
        e = jnp.where(e >= 0.0, e, e * 0.2)
        gid = ebase + j * CE + k * 16 + lax.iota(jnp.int32, 16)
        e = jnp.where(gid < E_TOT, e, NEG)
        e_v[j, sl] = e
        return jnp.maximum(vmax, e)

    vmax = lax.fori_loop(0, NBLK * 8, _e, jnp.full((16,), NEG, jnp.float32))
    mx_v[...] = vmax
    pltpu.sync_copy(mx_v, mx_sh.at[sid])
    plsc.subcore_barrier()

    pltpu.sync_copy(mx_sh, mx16_v)

    def _m(i, vm):
        return jnp.maximum(vm, mx16_v[i])
    g = jnp.max(lax.fori_loop(0, 16, _m, jnp.full((16,), NEG, jnp.float32)))

    def _p(i, _):
        j = i // 8
        k = i % 8
        sl = pl.ds(k * 16, 16)
        p = jnp.exp(e_v[j, sl] - g)
        e_v[j, sl] = p
        idst = dst_v[j, sl]
        plsc.addupdate_scatter(s_v, [idst >> 7, idst & 127], p)
        return 0
    lax.fori_loop(0, NBLK * 8, _p, 0)

    pltpu.sync_copy(s_v, s_sh.at[rid_v], add=True)
    plsc.subcore_barrier()
    pltpu.sync_copy(s_sh, s_v)

    def _a(i, _):
        j = i // 8
        k = i % 8
        sl = pl.ds(k * 16, 16)
        idst = dst_v[j, sl]
        s = plsc.load_gather(s_v, [idst >> 7, idst & 127])
        e_v[j, sl] = e_v[j, sl] / (s + 1e-16)
        return 0
    lax.fori_loop(0, NBLK * 8, _a, 0)

    half = NBLK // 2
    pltpu.sync_copy(e_v.at[pl.ds(cid * half, half)],
                    alpha_hbm.at[pl.ds(row0 + cid * half, half)])


_ZR = ACC_R // 16


@functools.partial(
    pl.kernel,
    out_type=jax.ShapeDtypeStruct((2 * NP, HD), jnp.float32),
    mesh=_mesh,
    compiler_params=_scp,
    scratch_types=[
        pltpu.VMEM((NBLK,), jnp.int32),
        pltpu.VMEM((NBLK, CE), jnp.int32),
        pltpu.VMEM((NBLK, CE), jnp.int32),
        pltpu.VMEM((NBLK, CE), jnp.int32),
        pltpu.VMEM((NBLK, CE), jnp.float32),
        pltpu.VMEM((CE, HD), jnp.float32),
        pltpu.VMEM((CE, HD), jnp.float32),
        pltpu.VMEM((41, HD), jnp.float32),
        pltpu.SemaphoreType.DMA,
        pltpu.VMEM_SHARED((ACC_R, HD), jnp.float32),
    ],
)
def _sc_agg(hh_hbm, src_hbm, dst_hbm, alpha_hbm, agg_hbm,
            rid_v, src_v, dst_v, dr_v, al_v, g0_v, g1_v, z_v, sem, acc_sh):
    sid = lax.axis_index("s")
    cid = lax.axis_index("c")
    row0 = sid * NBLK

    def _ri(i, _):
        rid_v[pl.ds(i * 16, 16)] = row0 + i * 16 + lax.iota(jnp.int32, 16)
        return 0
    lax.fori_loop(0, NBLK // 16, _ri, 0)
    pltpu.async_copy(src_hbm.at[rid_v], src_v, sem).wait()
    pltpu.async_copy(dst_hbm.at[rid_v], dst_v, sem).wait()
    pltpu.async_copy(alpha_hbm.at[rid_v], al_v, sem).wait()

    hoff = cid * NP

    def _sh(i, _):
        j = i // 8
        sl = pl.ds((i % 8) * 16, 16)
        src_v[j, sl] = src_v[j, sl] + hoff
        return 0
    lax.fori_loop(0, NBLK * 8, _sh, 0)

    def _z(i, _):
        z_v[i // 8, pl.ds((i % 8) * 16, 16)] = jnp.zeros((16,), jnp.float32)
        return 0
    lax.fori_loop(0, 41 * 8, _z, 0)

    def _pass(p, _):
        base = p * PASS_R

        def _zc(t, _):
            pltpu.sync_copy(z_v, acc_sh.at[pl.ds(sid * _ZR + t * 41, 41)])
            return 0
        lax.fori_loop(0, 8, _zc, 0)

        def _rm(i, _):
            j = i // 8
            sl = pl.ds((i % 8) * 16, 16)
            d = dst_v[j, sl] - base
            ok = (d >= 0) & (d < PASS_R)
            dr_v[j, sl] = jnp.where(ok, d, ACC_R - 1)
            return 0
        lax.fori_loop(0, NBLK * 8, _rm, 0)
        plsc.subcore_barrier()

        def _scale_scatter(j, g_v):
            def _row(r, _):
                zi = jnp.zeros((16,), jnp.int32)
                a = plsc.load_gather(al_v, [zi + j, zi + r])
                for k in range(8):
                    sl = pl.ds(k * 16, 16)
                    g_v[r, sl] = g_v[r, sl] * a
                return 0
            lax.fori_loop(0, CE, _row, 0)
            pltpu.sync_copy(g_v, acc_sh.at[dr_v.at[j]], add=True)

        pltpu.make_async_copy(hh_hbm.at[src_v.at[0]], g0_v, sem).start()

        def _blk2(j2, _):
            j = 2 * j2
            pltpu.make_async_copy(hh_hbm.at[src_v.at[j]], g0_v, sem).wait()
            pltpu.make_async_copy(hh_hbm.at[src_v.at[j + 1]], g1_v,
                                  sem).start()
            _scale_scatter(j, g0_v)
            pltpu.make_async_copy(hh_hbm.at[src_v.at[j + 1]], g1_v,
                                  sem).wait()

            @pl.when(j + 2 < NBLK)
            def _():
                pltpu.make_async_copy(hh_hbm.at[src_v.at[j + 2]], g0_v,
                                      sem).start()
            _scale_scatter(j + 1, g1_v)
            return 0

        lax.fori_loop(0, NBLK // 2, _blk2, 0)
        plsc.subcore_barrier()

        pltpu.sync_copy(
            acc_sh.at[pl.ds(sid * (PASS_R // 16), PASS_R // 16)],
            agg_hbm.at[pl.ds(cid * NP + base + sid * (PASS_R // 16),
                             PASS_R // 16)])
        plsc.subcore_barrier()
        return 0

    lax.fori_loop(0, NPASS, _pass, 0)


def _layer(hh, sc, src2, dst2):
    alpha = _sc_softmax(sc, src2, dst2)
    return _sc_agg(hh, src2, dst2, alpha)


def kernel(x, edge_index, batch, W1, att_src1, att_dst1, b1,
           W2, att_src2, att_dst2, b2, W3, att_src3, att_dst3, b3, Wf, bf):
    loop = jnp.arange(N, dtype=jnp.int32)
    pad = jnp.zeros((E_PAD - E_TOT,), jnp.int32)
    src2 = jnp.concatenate([edge_index[0].astype(jnp.int32), loop, pad]
                           ).reshape(EROWS, CE)
    dst2 = jnp.concatenate([edge_index[1].astype(jnp.int32), loop, pad]
                           ).reshape(EROWS, CE)
    batch3 = jnp.concatenate(
        [batch.astype(jnp.int32), jnp.full((NP - N,), NB, jnp.int32)]
    ).reshape(NP // _BMF, 1, _BMF)

    hh3, sc = _tc_in(x, W1, att_src1.reshape(1, DH), att_dst1.reshape(1, DH))
    agg = _layer(hh3.reshape(2 * NP, HD), sc, src2, dst2)
    hh3, sc = _tc_mid(agg, b1.reshape(1, DH), W2,
                      att_src2.reshape(1, DH), att_dst2.reshape(1, DH))
    agg = _layer(hh3.reshape(2 * NP, HD), sc, src2, dst2)
    hh3, sc = _tc_mid(agg, b2.reshape(1, DH), W3,
                      att_src3.reshape(1, DH), att_dst3.reshape(1, DH))
    agg = _layer(hh3.reshape(2 * NP, HD), sc, src2, dst2)
    return _tc_fin(agg, b3.reshape(1, DH), batch3, Wf, bf.reshape(1, DOUT))


# speedup vs baseline: 1.9416x; 1.9416x over previous
import functools

import jax
import jax.numpy as jnp
from jax import lax
from jax.experimental import pallas as pl
from jax.experimental.pallas import tpu as pltpu
from jax.experimental.pallas import tpu_sc as plsc

N = 10000
E = 160000
DIN = 256
DH = 256
DOUT = 128
NB = 64
HD = 128

E_TOT = E + N
NBLK = 96
CE = 128
T_E = NBLK * CE
E_PAD = 16 * T_E
EROWS = E_PAD // CE
SROWS = 80
NP = 10240
PASS_R = 5120
NPASS = 2
ACC_R = PASS_R + 128
NEG = -1e30

_mesh = plsc.VectorSubcoreMesh(core_axis_name="c", subcore_axis_name="s")
_scp = pltpu.CompilerParams(use_tc_tiling_on_sc=False,
                            needs_layout_passes=False)


def _tc_in_body(x_ref, w_ref, as_ref, ad_ref, hh_ref, sc_ref):
    h = lax.dot_general(x_ref[...], w_ref[...], (((1,), (1,)), ((), ())),
                        preferred_element_type=jnp.float32)
    hh_ref[0] = h[:, :HD]
    hh_ref[1] = h[:, HD:]
    s0 = jnp.sum(h * as_ref[...], axis=1, keepdims=True)
    s1 = jnp.sum(h * ad_ref[...], axis=1, keepdims=True)
    sc_ref[...] = jnp.concatenate([s0, s1], axis=1)


_BM = 1000


def _tc_in(x, W, a_s, a_d):
    return pl.pallas_call(
        _tc_in_body,
        grid=(N // _BM,),
        in_specs=[
            pl.BlockSpec((_BM, DIN), lambda i: (i, 0)),
            pl.BlockSpec((DH, DIN), lambda i: (0, 0)),
            pl.BlockSpec((1, DH), lambda i: (0, 0)),
            pl.BlockSpec((1, DH), lambda i: (0, 0)),
        ],
        out_specs=[
            pl.BlockSpec((2, _BM, HD), lambda i: (0, i, 0)),
            pl.BlockSpec((_BM, 2), lambda i: (i, 0)),
        ],
        out_shape=[
            jax.ShapeDtypeStruct((2, NP, HD), jnp.float32),
            jax.ShapeDtypeStruct((NP, 2), jnp.float32),
        ],
    )(x, W, a_s, a_d)


def _tc_mid_body(a0_ref, a1_ref, b_ref, w_ref, as_ref, ad_ref,
                 hh_ref, sc_ref):
    x0 = jnp.maximum(a0_ref[...] + b_ref[:, :HD], 0.0)
    x1 = jnp.maximum(a1_ref[...] + b_ref[:, HD:], 0.0)
    h = (lax.dot_general(x0, w_ref[:, :HD], (((1,), (1,)), ((), ())),
                         preferred_element_type=jnp.float32)
         + lax.dot_general(x1, w_ref[:, HD:], (((1,), (1,)), ((), ())),
                           preferred_element_type=jnp.float32))
    hh_ref[0] = h[:, :HD]
    hh_ref[1] = h[:, HD:]
    s0 = jnp.sum(h * as_ref[...], axis=1, keepdims=True)
    s1 = jnp.sum(h * ad_ref[...], axis=1, keepdims=True)
    sc_ref[...] = jnp.concatenate([s0, s1], axis=1)


_BM2 = 1024


def _tc_mid(agg, b, W, a_s, a_d):
    nb2 = NP // _BM2
    return pl.pallas_call(
        _tc_mid_body,
        grid=(nb2,),
        in_specs=[
            pl.BlockSpec((_BM2, HD), lambda i: (i, 0)),
            pl.BlockSpec((_BM2, HD), lambda i, nb2=nb2: (i + nb2, 0)),
            pl.BlockSpec((1, DH), lambda i: (0, 0)),
            pl.BlockSpec((DH, DH), lambda i: (0, 0)),
            pl.BlockSpec((1, DH), lambda i: (0, 0)),
            pl.BlockSpec((1, DH), lambda i: (0, 0)),
        ],
        out_specs=[
            pl.BlockSpec((2, _BM2, HD), lambda i: (0, i, 0)),
            pl.BlockSpec((_BM2, 2), lambda i: (i, 0)),
        ],
        out_shape=[
            jax.ShapeDtypeStruct((2, NP, HD), jnp.float32),
            jax.ShapeDtypeStruct((NP, 2), jnp.float32),
        ],
    )(agg, agg, b, W, a_s, a_d)


_BMF = 512


def _tc_fin_body(a0_ref, a1_ref, b_ref, bt_ref, wf_ref, bf_ref, o_ref,
                 acc_ref, cnt_ref):
    i = pl.program_id(0)

    @pl.when(i == 0)
    def _():
        acc_ref[...] = jnp.zeros_like(acc_ref)
        cnt_ref[...] = jnp.zeros_like(cnt_ref)

    x0 = jnp.maximum(a0_ref[...] + b_ref[:, :HD], 0.0)
    x1 = jnp.maximum(a1_ref[...] + b_ref[:, HD:], 0.0)
    bt = bt_ref[0]
    oh = (lax.broadcasted_iota(jnp.int32, (NB, _BMF), 0) == bt
          ).astype(jnp.float32)
    acc_ref[...] += jnp.concatenate(
        [lax.dot_general(oh, x0, (((1,), (0,)), ((), ())),
                         preferred_element_type=jnp.float32),
         lax.dot_general(oh, x1, (((1,), (0,)), ((), ())),
                         preferred_element_type=jnp.float32)], axis=1)
    cnt_ref[...] += lax.dot_general(oh, jnp.ones((_BMF, HD), jnp.float32),
                                    (((1,), (0,)), ((), ())),
                                    preferred_element_type=jnp.float32)

    @pl.when(i == pl.num_programs(0) - 1)
    def _():
        pooled = acc_ref[...] / jnp.maximum(cnt_ref[:, :1], 1.0)
        o_ref[...] = lax.dot_general(pooled, wf_ref[...],
                                     (((1,), (1,)), ((), ())),
                                     preferred_element_type=jnp.float32
                                     ) + bf_ref[...]


def _tc_fin(agg, b, batch3, Wf, bf):
    nbf = NP // _BMF
    return pl.pallas_call(
        _tc_fin_body,
        grid=(nbf,),
        in_specs=[
            pl.BlockSpec((_BMF, HD), lambda i: (i, 0)),
            pl.BlockSpec((_BMF, HD), lambda i, nbf=nbf: (i + nbf, 0)),
            pl.BlockSpec((1, DH), lambda i: (0, 0)),
            pl.BlockSpec((1, 1, _BMF), lambda i: (i, 0, 0)),
            pl.BlockSpec((DOUT, DH), lambda i: (0, 0)),
            pl.BlockSpec((1, DOUT), lambda i: (0, 0)),
        ],
        out_specs=pl.BlockSpec((NB, DOUT), lambda i: (0, 0)),
        out_shape=jax.ShapeDtypeStruct((NB, DOUT), jnp.float32),
        scratch_shapes=[
            pltpu.VMEM((NB, DH), jnp.float32),
            pltpu.VMEM((NB, HD), jnp.float32),
        ],
    )(agg, agg, b, batch3, Wf, bf)


@functools.partial(
    pl.kernel,
    out_type=jax.ShapeDtypeStruct((EROWS, CE), jnp.float32),
    mesh=_mesh,
    compiler_params=_scp,
    scratch_types=[
        pltpu.VMEM((NP, 2), jnp.float32),
        pltpu.VMEM((NBLK, CE), jnp.int32),
        pltpu.VMEM((NBLK, CE), jnp.int32),
        pltpu.VMEM((NBLK, CE), jnp.float32),
        pltpu.VMEM((SROWS, CE), jnp.float32),
        pltpu.VMEM((16,), jnp.float32),
        pltpu.VMEM((16, 16), jnp.float32),
        pltpu.VMEM((SROWS,), jnp.int32),
        pltpu.VMEM((NBLK,), jnp.int32),
        pltpu.SemaphoreType.DMA,
        pltpu.VMEM_SHARED((16, 16), jnp.float32),
        pltpu.VMEM_SHARED((SROWS, CE), jnp.float32),
    ],
)
def _sc_softmax(sc_hbm, src_hbm, dst_hbm, alpha_hbm,
                sc_v, src_v, dst_v, e_v, s_v, mx_v, mx16_v, rid_v, crid_v,
                sem, mx_sh, s_sh):
    sid = lax.axis_index("s")
    cid = lax.axis_index("c")
    row0 = sid * NBLK

    pltpu.sync_copy(sc_hbm, sc_v)

    def _cr(i, _):
        crid_v[pl.ds(i * 16, 16)] = row0 + i * 16 + lax.iota(jnp.int32, 16)
        return 0
    lax.fori_loop(0, NBLK // 16, _cr, 0)
    pltpu.async_copy(src_hbm.at[crid_v], src_v, sem).wait()
    pltpu.async_copy(dst_hbm.at[crid_v], dst_v, sem).wait()

    def _z(i, _):
        s_v[i // 8, pl.ds((i % 8) * 16, 16)] = jnp.zeros((16,), jnp.float32)
        return 0
    lax.fori_loop(0, SROWS * 8, _z, 0)

    def _r(i, _):
        rid_v[pl.ds(i * 16, 16)] = i * 16 + lax.iota(jnp.int32, 16)
        return 0
    lax.fori_loop(0, SROWS // 16, _r, 0)

    @pl.when(sid == 0)
    def _():
        pltpu.sync_copy(s_v, s_sh)

    ebase = sid * T_E

    def _e(i, vmax):
        j = i // 8
        k = i % 8
        sl = pl.ds(k * 16, 16)
        isrc = src_v[j, sl]
        idst = dst_v[j, sl]
        zi = jnp.zeros((16,), jnp.int32)
        e = (plsc.load_gather(sc_v, [isrc, zi])
             + plsc.load_gather(sc_v, [idst, zi + 1]))
        e = jnp.where(e >= 0.0, e, e * 0.2)
        gid = ebase + j * CE + k * 16 + lax.iota(jnp.int32, 16)
        e = jnp.where(gid < E_TOT, e, NEG)
        e_v[j, sl] = e
        return jnp.maximum(vmax, e)

    vmax = lax.fori_loop(0, NBLK * 8, _e, jnp.full((16,), NEG, jnp.float32))
    mx_v[...] = vmax
    pltpu.sync_copy(mx_v, mx_sh.at[sid])
    plsc.subcore_barrier()

    pltpu.sync_copy(mx_sh, mx16_v)

    def _m(i, vm):
        return jnp.maximum(vm, mx16_v[i])
    g = jnp.max(lax.fori_loop(0, 16, _m, jnp.full((16,), NEG, jnp.float32)))

    def _p(i, _):
        j = i // 8
        k = i % 8
        sl = pl.ds(k * 16, 16)
        p = jnp.exp(e_v[j, sl] - g)
        e_v[j, sl] = p
        idst = dst_v[j, sl]
        plsc.addupdate_scatter(s_v, [idst >> 7, idst & 127], p)
        return 0
    lax.fori_loop(0, NBLK * 8, _p, 0)

    pltpu.sync_copy(s_v, s_sh.at[rid_v], add=True)
    plsc.subcore_barrier()
    pltpu.sync_copy(s_sh, s_v)

    def _a(i, _):
        j = i // 8
        k = i % 8
        sl = pl.ds(k * 16, 16)
        idst = dst_v[j, sl]
        s = plsc.load_gather(s_v, [idst >> 7, idst & 127])
        e_v[j, sl] = e_v[j, sl] / (s + 1e-16)
        return 0
    lax.fori_loop(0, NBLK * 8, _a, 0)

    half = NBLK // 2
    pltpu.sync_copy(e_v.at[pl.ds(cid * half, half)],
                    alpha_hbm.at[pl.ds(row0 + cid * half, half)])


@functools.partial(
    pl.kernel,
    out_type=jax.ShapeDtypeStruct((2 * NP, HD), jnp.float32),
    mesh=_mesh,
    compiler_params=_scp,
    scratch_types=[
        pltpu.VMEM((NBLK,), jnp.int32),
        pltpu.VMEM((NBLK, CE), jnp.int32),
        pltpu.VMEM((NBLK, CE), jnp.int32),
        pltpu.VMEM((NBLK, CE), jnp.float32),
        pltpu.VMEM((64, HD), jnp.float32),
        pltpu.VMEM((8, HD), jnp.float32),
        pltpu.SemaphoreType.DMA,
        pltpu.VMEM_SHARED((NP, HD), jnp.float32),
    ],
)
def _sc_agg(hh_hbm, src_hbm, dst_hbm, alpha_hbm, agg_hbm,
            rid_v, src_v, dst_v, al_v, g_v, z_v, sem, acc_sh):
    sid = lax.axis_index("s")
    cid = lax.axis_index("c")
    row0 = sid * NBLK

    def _ri(i, _):
        rid_v[pl.ds(i * 16, 16)] = row0 + i * 16 + lax.iota(jnp.int32, 16)
        return 0
    lax.fori_loop(0, NBLK // 16, _ri, 0)
    pltpu.async_copy(src_hbm.at[rid_v], src_v, sem).wait()
    pltpu.async_copy(dst_hbm.at[rid_v], dst_v, sem).wait()
    pltpu.async_copy(alpha_hbm.at[rid_v], al_v, sem).wait()

    hoff = cid * NP

    def _sh(i, _):
        j = i // 8
        sl = pl.ds((i % 8) * 16, 16)
        src_v[j, sl] = src_v[j, sl] + hoff
        return 0
    lax.fori_loop(0, NBLK * 8, _sh, 0)

    def _z(i, _):
        z_v[i // 8, pl.ds((i % 8) * 16, 16)] = jnp.zeros((16,), jnp.float32)
        return 0
    lax.fori_loop(0, 8 * 8, _z, 0)

    def _zc(t, _):
        pltpu.sync_copy(z_v, acc_sh.at[pl.ds(sid * 640 + t * 8, 8)])
        return 0
    lax.fori_loop(0, 80, _zc, 0)
    plsc.subcore_barrier()

    def _half(i, _):
        j = i // 2
        h = i % 2
        pltpu.async_copy(hh_hbm.at[src_v.at[j, pl.ds(h * 64, 64)]],
                         g_v, sem).wait()

        def _row(r, _):
            zi = jnp.zeros((16,), jnp.int32)
            a = plsc.load_gather(al_v, [zi + j, zi + h * 64 + r])
            for k in range(8):
                sl = pl.ds(k * 16, 16)
                g_v[r, sl] = g_v[r, sl] * a
            return 0
        lax.fori_loop(0, 64, _row, 0)

        pltpu.sync_copy(g_v, acc_sh.at[dst_v.at[j, pl.ds(h * 64, 64)]],
                        add=True)
        return 0

    lax.fori_loop(0, NBLK * 2, _half, 0)
    plsc.subcore_barrier()

    pltpu.sync_copy(acc_sh.at[pl.ds(sid * 640, 640)],
                    agg_hbm.at[pl.ds(cid * NP + sid * 640, 640)])


def _layer(hh, sc, src2, dst2):
    alpha = _sc_softmax(sc, src2, dst2)
    return _sc_agg(hh, src2, dst2, alpha)


def kernel(x, edge_index, batch, W1, att_src1, att_dst1, b1,
           W2, att_src2, att_dst2, b2, W3, att_src3, att_dst3, b3, Wf, bf):
    loop = jnp.arange(N, dtype=jnp.int32)
    pad = jnp.zeros((E_PAD - E_TOT,), jnp.int32)
    src2 = jnp.concatenate([edge_index[0].astype(jnp.int32), loop, pad]
                           ).reshape(EROWS, CE)
    dst2 = jnp.concatenate([edge_index[1].astype(jnp.int32), loop, pad]
                           ).reshape(EROWS, CE)
    batch3 = jnp.concatenate(
        [batch.astype(jnp.int32), jnp.full((NP - N,), NB, jnp.int32)]
    ).reshape(NP // _BMF, 1, _BMF)

    hh3, sc = _tc_in(x, W1, att_src1.reshape(1, DH), att_dst1.reshape(1, DH))
    agg = _layer(hh3.reshape(2 * NP, HD), sc, src2, dst2)
    hh3, sc = _tc_mid(agg, b1.reshape(1, DH), W2,
                      att_src2.reshape(1, DH), att_dst2.reshape(1, DH))
    agg = _layer(hh3.reshape(2 * NP, HD), sc, src2, dst2)
    hh3, sc = _tc_mid(agg, b2.reshape(1, DH), W3,
                      att_src3.reshape(1, DH), att_dst3.reshape(1, DH))
    agg = _layer(hh3.reshape(2 * NP, HD), sc, src2, dst2)
    return _tc_fin(agg, b3.reshape(1, DH), batch3, Wf, bf.reshape(1, DOUT))


# speedup vs baseline: 2.2378x; 1.1526x over previous
import functools

import jax
import jax.numpy as jnp
from jax import lax
from jax.experimental import pallas as pl
from jax.experimental.pallas import tpu as pltpu
from jax.experimental.pallas import tpu_sc as plsc

N = 10000
E = 160000
DIN = 256
DH = 256
DOUT = 128
NB = 64

E_TOT = E + N
NBLK = 96
CE = 128
T_E = NBLK * CE
E_PAD = 16 * T_E
EROWS = E_PAD // CE
SROWS = 80
NP = 10240
HALF = 5120
ACC_R = HALF + 128
PADN = NP - 1
NEG = -1e30

_mesh = plsc.VectorSubcoreMesh(core_axis_name="c", subcore_axis_name="s")
_scp = pltpu.CompilerParams(use_tc_tiling_on_sc=False,
                            needs_layout_passes=False)


def _tc_in_body(x_ref, w_ref, as_ref, ad_ref, h_ref, sc_ref):
    h = lax.dot_general(x_ref[...], w_ref[...], (((1,), (1,)), ((), ())),
                        preferred_element_type=jnp.float32)
    h_ref[...] = h
    s0 = jnp.sum(h * as_ref[...], axis=1, keepdims=True)
    s1 = jnp.sum(h * ad_ref[...], axis=1, keepdims=True)
    sc_ref[...] = jnp.concatenate([s0, s1], axis=1)


_BM = 1000


def _tc_in(x, W, a_s, a_d):
    return pl.pallas_call(
        _tc_in_body,
        grid=(N // _BM,),
        in_specs=[
            pl.BlockSpec((_BM, DIN), lambda i: (i, 0)),
            pl.BlockSpec((DH, DIN), lambda i: (0, 0)),
            pl.BlockSpec((1, DH), lambda i: (0, 0)),
            pl.BlockSpec((1, DH), lambda i: (0, 0)),
        ],
        out_specs=[
            pl.BlockSpec((_BM, DH), lambda i: (i, 0)),
            pl.BlockSpec((_BM, 2), lambda i: (i, 0)),
        ],
        out_shape=[
            jax.ShapeDtypeStruct((NP, DH), jnp.float32),
            jax.ShapeDtypeStruct((NP, 2), jnp.float32),
        ],
    )(x, W, a_s, a_d)


def _tc_mid_body(a_ref, b_ref, w_ref, as_ref, ad_ref, h_ref, sc_ref):
    x = jnp.maximum(a_ref[...] + b_ref[...], 0.0)
    h = lax.dot_general(x, w_ref[...], (((1,), (1,)), ((), ())),
                        preferred_element_type=jnp.float32)
    h_ref[...] = h
    s0 = jnp.sum(h * as_ref[...], axis=1, keepdims=True)
    s1 = jnp.sum(h * ad_ref[...], axis=1, keepdims=True)
    sc_ref[...] = jnp.concatenate([s0, s1], axis=1)


_BM2 = 1024


def _tc_mid(agg, b, W, a_s, a_d):
    return pl.pallas_call(
        _tc_mid_body,
        grid=(NP // _BM2,),
        in_specs=[
            pl.BlockSpec((_BM2, DH), lambda i: (i, 0)),
            pl.BlockSpec((1, DH), lambda i: (0, 0)),
            pl.BlockSpec((DH, DH), lambda i: (0, 0)),
            pl.BlockSpec((1, DH), lambda i: (0, 0)),
            pl.BlockSpec((1, DH), lambda i: (0, 0)),
        ],
        out_specs=[
            pl.BlockSpec((_BM2, DH), lambda i: (i, 0)),
            pl.BlockSpec((_BM2, 2), lambda i: (i, 0)),
        ],
        out_shape=[
            jax.ShapeDtypeStruct((NP, DH), jnp.float32),
            jax.ShapeDtypeStruct((NP, 2), jnp.float32),
        ],
    )(agg, b, W, a_s, a_d)


_BMF = 512


def _tc_fin_body(a_ref, b_ref, bt_ref, wf_ref, bf_ref, o_ref,
                 acc_ref, cnt_ref):
    i = pl.program_id(0)

    @pl.when(i == 0)
    def _():
        acc_ref[...] = jnp.zeros_like(acc_ref)
        cnt_ref[...] = jnp.zeros_like(cnt_ref)

    x = jnp.maximum(a_ref[...] + b_ref[...], 0.0)
    bt = bt_ref[0]
    oh = (lax.broadcasted_iota(jnp.int32, (NB, _BMF), 0) == bt
          ).astype(jnp.float32)
    acc_ref[...] += lax.dot_general(oh, x, (((1,), (0,)), ((), ())),
                                    preferred_element_type=jnp.float32)
    cnt_ref[...] += lax.dot_general(oh, jnp.ones((_BMF, DOUT), jnp.float32),
                                    (((1,), (0,)), ((), ())),
                                    preferred_element_type=jnp.float32)

    @pl.when(i == pl.num_programs(0) - 1)
    def _():
        pooled = acc_ref[...] / jnp.maximum(cnt_ref[:, :1], 1.0)
        o_ref[...] = lax.dot_general(pooled, wf_ref[...],
                                     (((1,), (1,)), ((), ())),
                                     preferred_element_type=jnp.float32
                                     ) + bf_ref[...]


def _tc_fin(agg, b, batch3, Wf, bf):
    return pl.pallas_call(
        _tc_fin_body,
        grid=(NP // _BMF,),
        in_specs=[
            pl.BlockSpec((_BMF, DH), lambda i: (i, 0)),
            pl.BlockSpec((1, DH), lambda i: (0, 0)),
            pl.BlockSpec((1, 1, _BMF), lambda i: (i, 0, 0)),
            pl.BlockSpec((DOUT, DH), lambda i: (0, 0)),
            pl.BlockSpec((1, DOUT), lambda i: (0, 0)),
        ],
        out_specs=pl.BlockSpec((NB, DOUT), lambda i: (0, 0)),
        out_shape=jax.ShapeDtypeStruct((NB, DOUT), jnp.float32),
        scratch_shapes=[
            pltpu.VMEM((NB, DH), jnp.float32),
            pltpu.VMEM((NB, DOUT), jnp.float32),
        ],
    )(agg, b, batch3, Wf, bf)


@functools.partial(
    pl.kernel,
    out_type=(jax.ShapeDtypeStruct((EROWS, CE), jnp.int32),
              jax.ShapeDtypeStruct((EROWS, CE), jnp.int32),
              jax.ShapeDtypeStruct((16, 16), jnp.int32)),
    mesh=_mesh,
    compiler_params=_scp,
    scratch_types=[
        pltpu.VMEM((NBLK, CE), jnp.int32),
        pltpu.VMEM((NBLK, CE), jnp.int32),
        pltpu.VMEM((T_E + 16,), jnp.int32),
        pltpu.VMEM((T_E + 16,), jnp.int32),
        pltpu.VMEM((16,), jnp.int32),
        pltpu.VMEM((NBLK,), jnp.int32),
        pltpu.SemaphoreType.DMA,
    ],
)
def _sc_part(src_hbm, dst_hbm, srcp_hbm, dstp_hbm, cnt_hbm,
             src_v, dst_v, ps_v, pd_v, c_v, crid_v, sem):
    sid = lax.axis_index("s")
    cid = lax.axis_index("c")
    row0 = sid * NBLK

    def _cr(i, _):
        crid_v[pl.ds(i * 16, 16)] = row0 + i * 16 + lax.iota(jnp.int32, 16)
        return 0
    lax.fori_loop(0, NBLK // 16, _cr, 0)
    pltpu.async_copy(src_hbm.at[crid_v], src_v, sem).wait()
    pltpu.async_copy(dst_hbm.at[crid_v], dst_v, sem).wait()

    def _c(i, c):
        d = dst_v[i // 8, pl.ds((i % 8) * 16, 16)]
        return c + plsc.all_reduce_population_count(d < HALF)[0]
    nlo = lax.fori_loop(0, NBLK * 8, _c, jnp.int32(0))

    def _k(i, oo):
        olo, ohi = oo
        j = i // 8
        sl = pl.ds((i % 8) * 16, 16)
        sv = src_v[j, sl]
        dv = dst_v[j, sl]
        m = dv < HALF
        plsc.store_compressed(ps_v.at[pl.ds(olo, 16)], sv, mask=m)
        plsc.store_compressed(pd_v.at[pl.ds(olo, 16)], dv, mask=m)
        plsc.store_compressed(ps_v.at[pl.ds(ohi, 16)], sv, mask=~m)
        plsc.store_compressed(pd_v.at[pl.ds(ohi, 16)], dv, mask=~m)
        clo = plsc.all_reduce_population_count(m)[0]
        return olo + clo, ohi + (16 - clo)
    lax.fori_loop(0, NBLK * 8, _k, (jnp.int32(0), nlo))

    def _t(q, _):
        src_v[q // 8, pl.ds((q % 8) * 16, 16)] = ps_v[pl.ds(q * 16, 16)]
        dst_v[q // 8, pl.ds((q % 8) * 16, 16)] = pd_v[pl.ds(q * 16, 16)]
        return 0
    lax.fori_loop(0, NBLK * 8, _t, 0)
    c_v[...] = jnp.zeros((16,), jnp.int32) + nlo

    @pl.when(cid == 0)
    def _():
        pltpu.sync_copy(src_v, srcp_hbm.at[pl.ds(row0, NBLK)])
        pltpu.sync_copy(dst_v, dstp_hbm.at[pl.ds(row0, NBLK)])
        pltpu.sync_copy(c_v, cnt_hbm.at[sid])


@functools.partial(
    pl.kernel,
    out_type=jax.ShapeDtypeStruct((EROWS, CE), jnp.float32),
    mesh=_mesh,
    compiler_params=_scp,
    scratch_types=[
        pltpu.VMEM((2 * NP,), jnp.float32),
        pltpu.VMEM((NBLK, CE), jnp.int32),
        pltpu.VMEM((NBLK, CE), jnp.int32),
        pltpu.VMEM((NBLK, CE), jnp.float32),
        pltpu.VMEM((SROWS, CE), jnp.float32),
        pltpu.VMEM((16,), jnp.float32),
        pltpu.VMEM((16, 16), jnp.float32),
        pltpu.VMEM((SROWS,), jnp.int32),
        pltpu.VMEM((NBLK,), jnp.int32),
        pltpu.SemaphoreType.DMA,
        pltpu.VMEM_SHARED((16, 16), jnp.float32),
        pltpu.VMEM_SHARED((SROWS, CE), jnp.float32),
    ],
)
def _sc_softmax(sc_hbm, src_hbm, dst_hbm, alpha_hbm,
                sc_v, src_v, dst_v, e_v, s_v, mx_v, mx16_v, rid_v, crid_v,
                sem, mx_sh, s_sh):
    sid = lax.axis_index("s")
    cid = lax.axis_index("c")
    row0 = sid * NBLK

    pltpu.sync_copy(sc_hbm, sc_v)

    def _cl(i, _):
        sc_v[pl.ds(2 * N + i * 16, 16)] = jnp.zeros((16,), jnp.float32)
        return 0
    lax.fori_loop(0, 2 * (NP - N) // 16, _cl, 0)

    def _cr(i, _):
        crid_v[pl.ds(i * 16, 16)] = row0 + i * 16 + lax.iota(jnp.int32, 16)
        return 0
    lax.fori_loop(0, NBLK // 16, _cr, 0)
    pltpu.async_copy(src_hbm.at[crid_v], src_v, sem).wait()
    pltpu.async_copy(dst_hbm.at[crid_v], dst_v, sem).wait()

    def _z(i, _):
        s_v[i // 8, pl.ds((i % 8) * 16, 16)] = jnp.zeros((16,), jnp.float32)
        return 0
    lax.fori_loop(0, SROWS * 8, _z, 0)

    def _r(i, _):
        rid_v[pl.ds(i * 16, 16)] = i * 16 + lax.iota(jnp.int32, 16)
        return 0
    lax.fori_loop(0, SROWS // 16, _r, 0)

    @pl.when(sid == 0)
    def _():
        pltpu.sync_copy(s_v, s_sh)

    def _e(i, vmax):
        j = i // 8
        sl = pl.ds((i % 8) * 16, 16)
        e = (plsc.load_gather(sc_v, [src_v[j, sl] * 2])
             + plsc.load_gather(sc_v, [dst_v[j, sl] * 2 + 1]))
        e = jnp.where(e >= 0.0, e, e * 0.2)
        e_v[j, sl] = e
        return jnp.maximum(vmax, e)

    vmax = lax.fori_loop(0, NBLK * 8, _e, jnp.full((16,), NEG, jnp.float32))
    mx_v[...] = vmax
    pltpu.sync_copy(mx_v, mx_sh.at[sid])
    plsc.subcore_barrier()

    pltpu.sync_copy(mx_sh, mx16_v)

    def _m(i, vm):
        return jnp.maximum(vm, mx16_v[i])
    g = jnp.max(lax.fori_loop(0, 16, _m, jnp.full((16,), NEG, jnp.float32)))

    def _p(i, _):
        j = i // 8
        sl = pl.ds((i % 8) * 16, 16)
        p = jnp.exp(e_v[j, sl] - g)
        e_v[j, sl] = p
        idst = dst_v[j, sl]
        plsc.addupdate_scatter(s_v, [idst >> 7, idst & 127], p)
        return 0
    lax.fori_loop(0, NBLK * 8, _p, 0)

    pltpu.sync_copy(s_v, s_sh.at[rid_v], add=True)
    plsc.subcore_barrier()
    pltpu.sync_copy(s_sh, s_v)

    def _a(i, _):
        j = i // 8
        sl = pl.ds((i % 8) * 16, 16)
        idst = dst_v[j, sl]
        s = plsc.load_gather(s_v, [idst >> 7, idst & 127])
        e_v[j, sl] = e_v[j, sl] / (s + 1e-16)
        return 0
    lax.fori_loop(0, NBLK * 8, _a, 0)

    half = NBLK // 2
    pltpu.sync_copy(e_v.at[pl.ds(cid * half, half)],
                    alpha_hbm.at[pl.ds(row0 + cid * half, half)])


@functools.partial(
    pl.kernel,
    out_type=jax.ShapeDtypeStruct((NP, DH), jnp.float32),
    mesh=_mesh,
    compiler_params=_scp,
    scratch_types=[
        pltpu.VMEM((NBLK,), jnp.int32),
        pltpu.VMEM((NBLK, CE), jnp.int32),
        pltpu.VMEM((NBLK, CE), jnp.int32),
        pltpu.VMEM((NBLK, CE), jnp.float32),
        pltpu.VMEM((16, 16), jnp.int32),
        pltpu.VMEM((32, DH), jnp.float32),
        pltpu.VMEM((4, DH), jnp.float32),
        pltpu.SemaphoreType.DMA,
        pltpu.VMEM_SHARED((ACC_R, DH), jnp.float32),
    ],
)
def _sc_agg(h_hbm, src_hbm, dst_hbm, alpha_hbm, cnt_hbm, agg_hbm,
            rid_v, src_v, dst_v, al_v, cnt_v, g_v, z_v, sem, acc_sh):
    sid = lax.axis_index("s")
    cid = lax.axis_index("c")
    row0 = sid * NBLK

    def _ri(i, _):
        rid_v[pl.ds(i * 16, 16)] = row0 + i * 16 + lax.iota(jnp.int32, 16)
        return 0
    lax.fori_loop(0, NBLK // 16, _ri, 0)
    pltpu.async_copy(src_hbm.at[rid_v], src_v, sem).wait()
    pltpu.async_copy(dst_hbm.at[rid_v], dst_v, sem).wait()
    pltpu.async_copy(alpha_hbm.at[rid_v], al_v, sem).wait()
    pltpu.sync_copy(cnt_hbm, cnt_v)

    base = cid * HALF

    def _rm(i, _):
        j = i // 8
        sl = pl.ds((i % 8) * 16, 16)
        d = dst_v[j, sl] - base
        ok = (d >= 0) & (d < HALF)
        dst_v[j, sl] = jnp.where(ok, d, ACC_R - 1)
        return 0
    lax.fori_loop(0, NBLK * 8, _rm, 0)

    def _z(i, _):
        z_v[i // 16, pl.ds((i % 16) * 16, 16)] = jnp.zeros((16,), jnp.float32)
        return 0
    lax.fori_loop(0, 4 * (DH // 16), _z, 0)

    def _zc(t, _):
        pltpu.sync_copy(z_v, acc_sh.at[pl.ds(sid * (ACC_R // 16) + t * 4, 4)])
        return 0
    lax.fori_loop(0, ACC_R // 64, _zc, 0)
    plsc.subcore_barrier()

    nlo = cnt_v[sid][0]
    lo = jnp.where(cid == 0, 0, nlo >> 5)
    hi = jnp.where(cid == 0, (nlo + 31) >> 5, NBLK * 4)

    def _unit(i, _):
        j = i // 4
        q = i % 4
        pltpu.async_copy(h_hbm.at[src_v.at[j, pl.ds(q * 32, 32)]],
                         g_v, sem).wait()

        def _row(r, _):
            zi = jnp.zeros((16,), jnp.int32)
            a = plsc.load_gather(al_v, [zi + j, zi + q * 32 + r])
            for k in range(DH // 16):
                sl = pl.ds(k * 16, 16)
                g_v[r, sl] = g_v[r, sl] * a
            return 0
        lax.fori_loop(0, 32, _row, 0)

        pltpu.sync_copy(g_v, acc_sh.at[dst_v.at[j, pl.ds(q * 32, 32)]],
                        add=True)
        return 0

    lax.fori_loop(lo, hi, _unit, 0)
    plsc.subcore_barrier()

    pltpu.sync_copy(acc_sh.at[pl.ds(sid * (HALF // 16), HALF // 16)],
                    agg_hbm.at[pl.ds(base + sid * (HALF // 16), HALF // 16)])


def kernel(x, edge_index, batch, W1, att_src1, att_dst1, b1,
           W2, att_src2, att_dst2, b2, W3, att_src3, att_dst3, b3, Wf, bf):
    loop = jnp.arange(N, dtype=jnp.int32)
    pads = jnp.zeros((E_PAD - E_TOT,), jnp.int32)
    padd = jnp.full((E_PAD - E_TOT,), PADN, jnp.int32)
    src2 = jnp.concatenate([edge_index[0].astype(jnp.int32), loop, pads]
                           ).reshape(EROWS, CE)
    dst2 = jnp.concatenate([edge_index[1].astype(jnp.int32), loop, padd]
                           ).reshape(EROWS, CE)
    batch3 = jnp.concatenate(
        [batch.astype(jnp.int32), jnp.full((NP - N,), NB, jnp.int32)]
    ).reshape(NP // _BMF, 1, _BMF)

    srcp, dstp, cnt = _sc_part(src2, dst2)

    def layer(h, sc):
        alpha = _sc_softmax(sc.reshape(2 * NP), srcp, dstp)
        return _sc_agg(h, srcp, dstp, alpha, cnt)

    h, sc = _tc_in(x, W1, att_src1.reshape(1, DH), att_dst1.reshape(1, DH))
    agg = layer(h, sc)
    h, sc = _tc_mid(agg, b1.reshape(1, DH), W2,
                    att_src2.reshape(1, DH), att_dst2.reshape(1, DH))
    agg = layer(h, sc)
    h, sc = _tc_mid(agg, b2.reshape(1, DH), W3,
                    att_src3.reshape(1, DH), att_dst3.reshape(1, DH))
    agg = layer(h, sc)
    return _tc_fin(agg, b3.reshape(1, DH), batch3, Wf, bf.reshape(1, DOUT))
